# baseline (speedup 1.0000x reference)
import functools

import jax
import jax.numpy as jnp
from jax import lax
from jax.experimental import pallas as pl
from jax.experimental.pallas import tpu as pltpu
from jax.experimental.pallas import tpu_sc as plsc

_N = 10000
_E = 320000
_D = 128
_H = 64
_O = 64
_L = 5
_G = 64
_EMB = 320

_W = 128
_NC = 2
_NS = 16
_NW = _NC * _NS
_EPT = _E // _NW
_CHUNK = 80
_NCHUNK = _EPT // _CHUNK
_ITILES = 10
_RPT = _N // _ITILES
_RSTG = 200


def _sc_scatter(hp, src, dst, zrows):
    mesh = plsc.VectorSubcoreMesh(core_axis_name="c", subcore_axis_name="s")

    @functools.partial(
        pl.kernel,
        mesh=mesh,
        out_type=jax.ShapeDtypeStruct((_NC * _N, _W), jnp.float32),
        scratch_types=[
            pltpu.VMEM_SHARED((_N, _W), jnp.float32),
            pltpu.VMEM((_CHUNK,), jnp.int32),
            pltpu.VMEM((_CHUNK,), jnp.int32),
            pltpu.VMEM((_CHUNK, _W), jnp.float32),
            pltpu.VMEM((_RSTG, _W), jnp.float32),
            pltpu.SemaphoreType.DMA,
        ],
    )
    def k(hp_hbm, src_hbm, dst_hbm, z_hbm, out_hbm, acc, src_v, dst_v, rows_v, stage_v, sem):
        cid = lax.axis_index("c")
        sid = lax.axis_index("s")
        wid = sid * _NC + cid

        @pl.when(sid < _ITILES)
        def _init():
            pltpu.sync_copy(z_hbm, stage_v)
            for j in range(_RPT // _RSTG):
                r0 = sid * _RPT + j * _RSTG
                pltpu.sync_copy(stage_v, acc.at[pl.ds(r0, _RSTG)])

        plsc.subcore_barrier()

        def body(g, carry):
            off = pl.multiple_of(wid * _EPT + g * _CHUNK, 8)
            pltpu.sync_copy(src_hbm.at[pl.ds(off, _CHUNK)], src_v)
            pltpu.sync_copy(dst_hbm.at[pl.ds(off, _CHUNK)], dst_v)
            pltpu.async_copy(hp_hbm.at[src_v], rows_v, sem).wait()
            pltpu.sync_copy(rows_v, acc.at[dst_v], add=True)
            return carry

        lax.fori_loop(0, _NCHUNK, body, 0)
        plsc.subcore_barrier()

        @pl.when(sid < _ITILES)
        def _writeout():
            for j in range(_RPT // _RSTG):
                r0 = sid * _RPT + j * _RSTG
                pltpu.sync_copy(acc.at[pl.ds(r0, _RSTG)], stage_v)
                pltpu.sync_copy(stage_v, out_hbm.at[pl.ds(cid * _N + r0, _RSTG)])

    return k(hp, src, dst, zrows)


def _gin_t1_body(fin, h_ref, agg_ref, sv_ref, w1_ref, b1_ref, t1_ref):
    u = h_ref[:, 0:fin] * sv_ref[...] + agg_ref[0:_N, 0:fin] + agg_ref[_N:2 * _N, 0:fin]
    t1_ref[...] = jnp.dot(u, w1_ref[...], preferred_element_type=jnp.float32) + b1_ref[...]


def _bn_mm_body(t_ref, m_ref, v_ref, g_ref, b_ref, w2_ref, b2_ref, t2_ref):
    s = jnp.maximum(
        (t_ref[...] - m_ref[...]) / jnp.sqrt(v_ref[...] + 1e-5) * g_ref[...] + b_ref[...],
        0.0)
    t2_ref[...] = jnp.dot(s, w2_ref[...], preferred_element_type=jnp.float32) + b2_ref[...]


def _bn_out_body(has_next, t_ref, m_ref, v_ref, g_ref, b_ref, xs_ref, hn_ref=None):
    h = jnp.maximum(
        (t_ref[...] - m_ref[...]) / jnp.sqrt(v_ref[...] + 1e-5) * g_ref[...] + b_ref[...],
        0.0)
    xs_ref[...] = h
    if has_next:
        hn_ref[...] = jnp.concatenate([h, jnp.zeros((_N, _W - _H), jnp.float32)], axis=1)


def _layer_update(h_pad, agg, eps, conv, bn, has_next):
    fin = conv["W1"].shape[0]
    sv = jnp.full((1, fin), 1.0 + eps, jnp.float32)
    nh = jax.ShapeDtypeStruct((_N, _H), jnp.float32)
    t1 = pl.pallas_call(
        functools.partial(_gin_t1_body, fin), out_shape=nh,
    )(h_pad, agg, sv, conv["W1"], conv["b1"].reshape(1, _H))
    m1, v1 = t1.mean(0), t1.var(0)
    t2 = pl.pallas_call(_bn_mm_body, out_shape=nh)(
        t1, m1.reshape(1, _H), v1.reshape(1, _H),
        conv["g1"].reshape(1, _H), conv["be1"].reshape(1, _H),
        conv["W2"], conv["b2"].reshape(1, _H))
    m2, v2 = t2.mean(0), t2.var(0)
    out_shape = [nh]
    if has_next:
        out_shape.append(jax.ShapeDtypeStruct((_N, _W), jnp.float32))
    res = pl.pallas_call(
        functools.partial(_bn_out_body, has_next), out_shape=out_shape,
    )(t2, m2.reshape(1, _H), v2.reshape(1, _H),
      bn["g"].reshape(1, _H), bn["b"].reshape(1, _H))
    return res if has_next else (res[0], None)


def _pool_body(b_ref, x0, x1, x2, x3, x4, w0, w1, w2, w3, w4, bb_ref,
               gw1, gb1, gw2, gb2, gw3, gb3, gws, gbs, xc_ref, ge_ref):
    iota = lax.broadcasted_iota(jnp.int32, (1, _G), 1)
    oh = (b_ref[...] == iota).astype(jnp.float32)
    dn = (((0,), (0,)), ((), ()))
    ps = []
    for xr, wr in ((x0, w0), (x1, w1), (x2, w2), (x3, w3), (x4, w4)):
        p = lax.dot_general(oh, xr[...], dn, preferred_element_type=jnp.float32)
        ps.append(jnp.dot(p, wr[...], preferred_element_type=jnp.float32))
    xc = jnp.concatenate(ps, axis=1) + bb_ref[...]
    xc_ref[...] = xc
    h = jnp.maximum(jnp.dot(xc, gw1[...], preferred_element_type=jnp.float32) + gb1[...], 0.0)
    h = jnp.maximum(jnp.dot(h, gw2[...], preferred_element_type=jnp.float32) + gb2[...], 0.0)
    h = jnp.maximum(jnp.dot(h, gw3[...], preferred_element_type=jnp.float32) + gb3[...], 0.0)
    ge_ref[...] = h + jnp.dot(xc, gws[...], preferred_element_type=jnp.float32) + gbs[...]


def _pool_and_global(batch2d, xs, pred, gd):
    bb = jnp.concatenate([p["b"] for p in pred]).reshape(1, _EMB)
    args = [batch2d] + list(xs) + [p["W"] for p in pred] + [
        bb,
        gd["W1"], gd["b1"].reshape(1, _EMB),
        gd["W2"], gd["b2"].reshape(1, _EMB),
        gd["W3"], gd["b3"].reshape(1, _EMB),
        gd["Ws"], gd["bs"].reshape(1, _EMB),
    ]
    return pl.pallas_call(
        _pool_body,
        out_shape=(
            jax.ShapeDtypeStruct((_G, _EMB), jnp.float32),
            jax.ShapeDtypeStruct((_G, _EMB), jnp.float32),
        ),
    )(*args)


_BLK = 1000


def _node_body(x0, x1, x2, x3, x4, w1, b1, w2, b2, w3, b3, ws, bs, o_ref):
    z = jnp.concatenate([x0[...], x1[...], x2[...], x3[...], x4[...]], axis=1)
    h = jnp.maximum(jnp.dot(z, w1[...], preferred_element_type=jnp.float32) + b1[...], 0.0)
    h = jnp.maximum(jnp.dot(h, w2[...], preferred_element_type=jnp.float32) + b2[...], 0.0)
    h = jnp.maximum(jnp.dot(h, w3[...], preferred_element_type=jnp.float32) + b3[...], 0.0)
    o_ref[...] = h + jnp.dot(z, ws[...], preferred_element_type=jnp.float32) + bs[...]


def _node_embed(xs, ld):
    xspec = pl.BlockSpec((_BLK, _H), lambda i: (i, 0))
    wspec = pl.BlockSpec((_EMB, _EMB), lambda i: (0, 0))
    bspec = pl.BlockSpec((1, _EMB), lambda i: (0, 0))
    args = list(xs) + [
        ld["W1"], ld["b1"].reshape(1, _EMB),
        ld["W2"], ld["b2"].reshape(1, _EMB),
        ld["W3"], ld["b3"].reshape(1, _EMB),
        ld["Ws"], ld["bs"].reshape(1, _EMB),
    ]
    return pl.pallas_call(
        _node_body,
        grid=(_N // _BLK,),
        in_specs=[xspec] * 5 + [wspec, bspec] * 4,
        out_specs=pl.BlockSpec((_BLK, _EMB), lambda i: (i, 0)),
        out_shape=jax.ShapeDtypeStruct((_N, _EMB), jnp.float32),
    )(*args)


def kernel(x, edge_index, batch, params):
    src = edge_index[0]
    dst = edge_index[1]
    convs = params["convs"]
    eps = params["eps"]

    order = jnp.argsort(dst, stable=True)
    srcs = src[order]
    dsts = dst[order]
    zrows = jnp.zeros((_RSTG, _W), jnp.float32)

    h_pad = x
    xs = []
    for i in range(_L):
        agg = _sc_scatter(h_pad, srcs, dsts, zrows)
        has_next = i + 1 < _L
        h, h_pad = _layer_update(h_pad, agg, eps[i], convs[i], params["bns"][i], has_next)
        xs.append(h)

    xc, gemb = _pool_and_global(batch.reshape(_N, 1), xs, params["pred"], params["global_d"])
    nemb = _node_embed(xs, params["local_d"])
    return (gemb, nemb, xc)


# speedup vs baseline: 1.4860x; 1.4860x over previous
import functools

import jax
import jax.numpy as jnp
from jax import lax
from jax.experimental import pallas as pl
from jax.experimental.pallas import tpu as pltpu
from jax.experimental.pallas import tpu_sc as plsc

_N = 10000
_E = 320000
_D = 128
_H = 64
_O = 64
_L = 5
_G = 64
_EMB = 320

_W = 128
_NC = 2
_NS = 16
_NW = _NC * _NS
_CHUNK = 128
_NROWS = _E // _CHUNK
_TR = 80
_HTR = 40
_NROWS_PAD = _NW * _TR
_ITILES = 10
_RPT = _N // _ITILES
_RSTG = 40


def _sc_scatter(hp, src, dst, zrows):
    mesh = plsc.VectorSubcoreMesh(core_axis_name="c", subcore_axis_name="s")

    @functools.partial(
        pl.kernel,
        mesh=mesh,
        out_type=jax.ShapeDtypeStruct((_NC * _N, _W), jnp.float32),
        scratch_types=[
            pltpu.VMEM_SHARED((_N, _W), jnp.float32),
            pltpu.VMEM((_HTR, _CHUNK), jnp.int32),
            pltpu.VMEM((_HTR, _CHUNK), jnp.int32),
            pltpu.VMEM((_CHUNK, _W), jnp.float32),
            pltpu.VMEM((_CHUNK, _W), jnp.float32),
            pltpu.SemaphoreType.DMA,
            pltpu.SemaphoreType.DMA,
        ],
    )
    def k(hp_hbm, src_hbm, dst_hbm, z_hbm, out_hbm, acc, src_v, dst_v,
          rows0, rows1, sem0, sem1):
        cid = lax.axis_index("c")
        sid = lax.axis_index("s")
        wid = sid * _NC + cid

        row0 = pl.multiple_of(wid * _TR, 8)
        nrows = jnp.clip(_NROWS - wid * _TR, 0, _TR)

        @pl.when(sid < _ITILES)
        def _init():
            for j in range(_RPT // _RSTG):
                r0 = sid * _RPT + j * _RSTG
                pltpu.sync_copy(z_hbm, acc.at[pl.ds(r0, _RSTG)])

        plsc.subcore_barrier()

        bufs = (rows0, rows1)
        sems = (sem0, sem1)

        for half in range(_TR // _HTR):
            base = half * _HTR
            hrow = pl.multiple_of(row0 + base, 8)
            pltpu.async_copy(src_hbm.at[pl.ds(hrow, _HTR)], src_v, sem0)
            pltpu.async_copy(dst_hbm.at[pl.ds(hrow, _HTR)], dst_v, sem1).wait()
            pltpu.make_async_copy(src_hbm.at[pl.ds(hrow, _HTR)], src_v, sem0).wait()

            def start_gather(g, b):
                pltpu.async_copy(hp_hbm.at[src_v.at[g]], bufs[b], sems[b])

            def step(g, b):
                @pl.when(base + g < nrows)
                def _():
                    pltpu.make_async_copy(hp_hbm.at[src_v.at[g]], bufs[b], sems[b]).wait()

                    @pl.when(jnp.logical_and(g + 1 < _HTR, base + g + 1 < nrows))
                    def _():
                        start_gather(g + 1, 1 - b)

                    pltpu.sync_copy(bufs[b], acc.at[dst_v.at[g]], add=True)

            @pl.when(base < nrows)
            def _():
                start_gather(0, 0)

            def body(k2, carry):
                step(2 * k2, 0)
                step(2 * k2 + 1, 1)
                return carry

            lax.fori_loop(0, _HTR // 2, body, 0)

        plsc.subcore_barrier()

        @pl.when(sid < _ITILES)
        def _writeout():
            for j in range(_RPT // _RSTG):
                r0 = sid * _RPT + j * _RSTG
                pltpu.sync_copy(acc.at[pl.ds(r0, _RSTG)],
                                out_hbm.at[pl.ds(cid * _N + r0, _RSTG)])

    return k(hp, src, dst, zrows)


def _gin_t1_body(fin, h_ref, agg_ref, sv_ref, w1_ref, b1_ref, t1_ref):
    u = h_ref[:, 0:fin] * sv_ref[...] + agg_ref[0:_N, 0:fin] + agg_ref[_N:2 * _N, 0:fin]
    t1_ref[...] = jnp.dot(u, w1_ref[...], preferred_element_type=jnp.float32) + b1_ref[...]


def _bn_mm_body(t_ref, m_ref, v_ref, g_ref, b_ref, w2_ref, b2_ref, t2_ref):
    s = jnp.maximum(
        (t_ref[...] - m_ref[...]) / jnp.sqrt(v_ref[...] + 1e-5) * g_ref[...] + b_ref[...],
        0.0)
    t2_ref[...] = jnp.dot(s, w2_ref[...], preferred_element_type=jnp.float32) + b2_ref[...]


def _bn_out_body(has_next, t_ref, m_ref, v_ref, g_ref, b_ref, xs_ref, hn_ref=None):
    h = jnp.maximum(
        (t_ref[...] - m_ref[...]) / jnp.sqrt(v_ref[...] + 1e-5) * g_ref[...] + b_ref[...],
        0.0)
    xs_ref[...] = h
    if has_next:
        hn_ref[...] = jnp.concatenate([h, jnp.zeros((_N, _W - _H), jnp.float32)], axis=1)


def _layer_update(h_pad, agg, eps, conv, bn, has_next):
    fin = conv["W1"].shape[0]
    sv = jnp.full((1, fin), 1.0 + eps, jnp.float32)
    nh = jax.ShapeDtypeStruct((_N, _H), jnp.float32)
    t1 = pl.pallas_call(
        functools.partial(_gin_t1_body, fin), out_shape=nh,
    )(h_pad, agg, sv, conv["W1"], conv["b1"].reshape(1, _H))
    m1, v1 = t1.mean(0), t1.var(0)
    t2 = pl.pallas_call(_bn_mm_body, out_shape=nh)(
        t1, m1.reshape(1, _H), v1.reshape(1, _H),
        conv["g1"].reshape(1, _H), conv["be1"].reshape(1, _H),
        conv["W2"], conv["b2"].reshape(1, _H))
    m2, v2 = t2.mean(0), t2.var(0)
    out_shape = [nh]
    if has_next:
        out_shape.append(jax.ShapeDtypeStruct((_N, _W), jnp.float32))
    res = pl.pallas_call(
        functools.partial(_bn_out_body, has_next), out_shape=out_shape,
    )(t2, m2.reshape(1, _H), v2.reshape(1, _H),
      bn["g"].reshape(1, _H), bn["b"].reshape(1, _H))
    return res if has_next else (res[0], None)


def _pool_body(b_ref, x0, x1, x2, x3, x4, w0, w1, w2, w3, w4, bb_ref,
               gw1, gb1, gw2, gb2, gw3, gb3, gws, gbs, xc_ref, ge_ref):
    iota = lax.broadcasted_iota(jnp.int32, (1, _G), 1)
    oh = (b_ref[...] == iota).astype(jnp.float32)
    dn = (((0,), (0,)), ((), ()))
    ps = []
    for xr, wr in ((x0, w0), (x1, w1), (x2, w2), (x3, w3), (x4, w4)):
        p = lax.dot_general(oh, xr[...], dn, preferred_element_type=jnp.float32)
        ps.append(jnp.dot(p, wr[...], preferred_element_type=jnp.float32))
    xc = jnp.concatenate(ps, axis=1) + bb_ref[...]
    xc_ref[...] = xc
    h = jnp.maximum(jnp.dot(xc, gw1[...], preferred_element_type=jnp.float32) + gb1[...], 0.0)
    h = jnp.maximum(jnp.dot(h, gw2[...], preferred_element_type=jnp.float32) + gb2[...], 0.0)
    h = jnp.maximum(jnp.dot(h, gw3[...], preferred_element_type=jnp.float32) + gb3[...], 0.0)
    ge_ref[...] = h + jnp.dot(xc, gws[...], preferred_element_type=jnp.float32) + gbs[...]


def _pool_and_global(batch2d, xs, pred, gd):
    bb = jnp.concatenate([p["b"] for p in pred]).reshape(1, _EMB)
    args = [batch2d] + list(xs) + [p["W"] for p in pred] + [
        bb,
        gd["W1"], gd["b1"].reshape(1, _EMB),
        gd["W2"], gd["b2"].reshape(1, _EMB),
        gd["W3"], gd["b3"].reshape(1, _EMB),
        gd["Ws"], gd["bs"].reshape(1, _EMB),
    ]
    return pl.pallas_call(
        _pool_body,
        out_shape=(
            jax.ShapeDtypeStruct((_G, _EMB), jnp.float32),
            jax.ShapeDtypeStruct((_G, _EMB), jnp.float32),
        ),
    )(*args)


_BLK = 1000


def _node_body(x0, x1, x2, x3, x4, w1, b1, w2, b2, w3, b3, ws, bs, o_ref):
    z = jnp.concatenate([x0[...], x1[...], x2[...], x3[...], x4[...]], axis=1)
    h = jnp.maximum(jnp.dot(z, w1[...], preferred_element_type=jnp.float32) + b1[...], 0.0)
    h = jnp.maximum(jnp.dot(h, w2[...], preferred_element_type=jnp.float32) + b2[...], 0.0)
    h = jnp.maximum(jnp.dot(h, w3[...], preferred_element_type=jnp.float32) + b3[...], 0.0)
    o_ref[...] = h + jnp.dot(z, ws[...], preferred_element_type=jnp.float32) + bs[...]


def _node_embed(xs, ld):
    xspec = pl.BlockSpec((_BLK, _H), lambda i: (i, 0))
    wspec = pl.BlockSpec((_EMB, _EMB), lambda i: (0, 0))
    bspec = pl.BlockSpec((1, _EMB), lambda i: (0, 0))
    args = list(xs) + [
        ld["W1"], ld["b1"].reshape(1, _EMB),
        ld["W2"], ld["b2"].reshape(1, _EMB),
        ld["W3"], ld["b3"].reshape(1, _EMB),
        ld["Ws"], ld["bs"].reshape(1, _EMB),
    ]
    return pl.pallas_call(
        _node_body,
        grid=(_N // _BLK,),
        in_specs=[xspec] * 5 + [wspec, bspec] * 4,
        out_specs=pl.BlockSpec((_BLK, _EMB), lambda i: (i, 0)),
        out_shape=jax.ShapeDtypeStruct((_N, _EMB), jnp.float32),
    )(*args)


def kernel(x, edge_index, batch, params):
    src = edge_index[0]
    dst = edge_index[1]
    convs = params["convs"]
    eps = params["eps"]

    order = jnp.argsort(dst, stable=True)
    pad = jnp.zeros((_NROWS_PAD - _NROWS, _CHUNK), jnp.int32)
    srcs = jnp.concatenate([src[order].reshape(_NROWS, _CHUNK), pad], axis=0)
    dsts = jnp.concatenate([dst[order].reshape(_NROWS, _CHUNK), pad], axis=0)
    zrows = jnp.zeros((_RSTG, _W), jnp.float32)

    h_pad = x
    xs = []
    for i in range(_L):
        agg = _sc_scatter(h_pad, srcs, dsts, zrows)
        has_next = i + 1 < _L
        h, h_pad = _layer_update(h_pad, agg, eps[i], convs[i], params["bns"][i], has_next)
        xs.append(h)

    xc, gemb = _pool_and_global(batch.reshape(_N, 1), xs, params["pred"], params["global_d"])
    nemb = _node_embed(xs, params["local_d"])
    return (gemb, nemb, xc)
